# baseline (speedup 1.0000x reference)
import functools

import jax
import jax.numpy as jnp
from jax import lax
from jax.experimental import pallas as pl
from jax.experimental.pallas import tpu as pltpu, tpu_sc as plsc

_EPS = 1e-5


def _vq_tc_body(x_ref, w_ref, idx_ref, loss_ref, w2_ref):
    G, TB, C = x_ref.shape
    x = x_ref[...].reshape(G * TB, C)
    w = w_ref[...]
    T, C = x.shape
    K = w.shape[0]

    @pl.when(pl.program_id(0) == 0)
    def _():
        w2_ref[...] = jnp.sum(w * w, axis=1)

    w2 = w2_ref[...]
    x2 = jnp.sum(x * x, axis=1)
    s = lax.dot_general(x, w, (((1,), (1,)), ((), ())),
                        preferred_element_type=jnp.float32)
    d2 = x2[:, None] + w2[None, :] - 2.0 * s
    idx = jnp.argmin(d2, axis=1).astype(jnp.int32)
    d2m = jnp.min(d2, axis=1)
    oh = lax.broadcasted_iota(jnp.int32, (T, K), 1) == idx[:, None]
    w2s = jnp.sum(jnp.where(oh, w2[None, :], 0.0), axis=1)
    qx = 0.5 * (x2 + w2s - d2m)
    nx = jnp.maximum(jnp.sqrt(x2), _EPS)
    nq = jnp.maximum(jnp.sqrt(w2s), _EPS)
    sse = w2s / (nq * nq) + x2 / (nx * nx) - 2.0 * qx / (nq * nx)
    lsum = jnp.sum(sse.reshape(G, TB), axis=1) * (1.0 / (TB * C))
    loss_ref[...] = jnp.broadcast_to(lsum[:, None, None], (G, 1, 128))
    i = pl.program_id(0)
    idx_ref[pl.ds(i * T, T)] = idx


def _vq_scores(x, W, off, nb, G=2):
    B, T, C = x.shape
    K = W.shape[0]
    assert nb % G == 0 and off % G == 0
    return pl.pallas_call(
        _vq_tc_body,
        grid=(nb // G,),
        in_specs=[
            pl.BlockSpec((G, T, C), lambda i: (i + off // G, 0, 0)),
            pl.BlockSpec((K, C), lambda i: (0, 0)),
        ],
        out_specs=[
            pl.BlockSpec((nb * T,), lambda i: (0,)),
            pl.BlockSpec((G, 1, 128), lambda i: (i, 0, 0)),
        ],
        out_shape=[
            jax.ShapeDtypeStruct((nb * T,), jnp.int32),
            jax.ShapeDtypeStruct((nb, 1, 128), jnp.float32),
        ],
        scratch_shapes=[pltpu.VMEM((K,), jnp.float32)],
    )(x, W)


def _sc_gather(W, idx_flat, n_out, row0):
    N = idx_flat.shape[0]
    K, C = W.shape
    info = plsc.get_sparse_core_info()
    nw = info.num_cores * info.num_subcores
    assert N % (8 * nw) == 0 and row0 % 8 == 0, (N, nw, row0)
    b_per_w = N // nw
    mesh = plsc.VectorSubcoreMesh(core_axis_name="c", subcore_axis_name="s")

    @functools.partial(
        pl.kernel, mesh=mesh,
        out_type=jax.ShapeDtypeStruct((n_out, C), jnp.float32),
        scratch_types=[
            pltpu.VMEM((b_per_w,), jnp.int32),
            pltpu.VMEM((b_per_w, C), jnp.float32),
            pltpu.SemaphoreType.DMA,
        ],
    )
    def gather_k(table_hbm, idx_hbm, out_hbm, idx_v, rows_v, sem):
        wid = lax.axis_index("s") * info.num_cores + lax.axis_index("c")
        base = wid * b_per_w
        pltpu.sync_copy(idx_hbm.at[pl.ds(base, b_per_w)], idx_v)
        pltpu.async_copy(table_hbm.at[idx_v], rows_v, sem).wait()
        pltpu.sync_copy(rows_v, out_hbm.at[pl.ds(row0 + base, b_per_w)])

    return gather_k(W, idx_flat)


def kernel(x, W):
    B, T, C = x.shape
    h = B // 2
    idx_a, loss_a = _vq_scores(x, W, 0, h)
    q_full = _sc_gather(W, idx_a, B * T, 0)
    idx_b, loss_b = _vq_scores(x, W, h, B - h)
    q_b = _sc_gather(W, idx_b, (B - h) * T, 0)
    quant = lax.dynamic_update_slice(q_full, q_b, (h * T, 0))
    loss = jnp.concatenate([loss_a, loss_b], axis=0)[:, 0, 0]
    quant_st = quant.reshape(B, T, C)
    indices = jnp.concatenate([idx_a, idx_b], axis=0).reshape(B, T)
    return quant_st, loss, loss, indices


# speedup vs baseline: 1.1830x; 1.1830x over previous
import functools

import jax
import jax.numpy as jnp
from jax import lax
from jax.experimental import pallas as pl
from jax.experimental.pallas import tpu as pltpu, tpu_sc as plsc

_EPS = 1e-5


def _vq_tc_body(x_ref, w_ref, idx_ref, loss_ref):
    G, TB, C = x_ref.shape
    x = x_ref[...].reshape(G * TB, C)
    w = w_ref[...]
    T, C = x.shape
    K = w.shape[0]
    w2 = jnp.sum(w * w, axis=1)
    x2 = jnp.sum(x * x, axis=1)
    s = lax.dot_general(x, w, (((1,), (1,)), ((), ())),
                        preferred_element_type=jnp.float32)
    d2 = x2[:, None] + w2[None, :] - 2.0 * s
    idx = jnp.argmin(d2, axis=1).astype(jnp.int32)
    d2m = jnp.min(d2, axis=1)
    oh = lax.broadcasted_iota(jnp.int32, (T, K), 1) == idx[:, None]
    w2s = jnp.sum(jnp.where(oh, w2[None, :], 0.0), axis=1)
    qx = 0.5 * (x2 + w2s - d2m)
    nx = jnp.maximum(jnp.sqrt(x2), _EPS)
    nq = jnp.maximum(jnp.sqrt(w2s), _EPS)
    sse = w2s / (nq * nq) + x2 / (nx * nx) - 2.0 * qx / (nq * nx)
    lsum = jnp.sum(sse.reshape(G, TB), axis=1) * (1.0 / (TB * C))
    loss_ref[...] = jnp.broadcast_to(lsum[:, None, None], (G, 1, 128))
    i = pl.program_id(0)
    idx_ref[pl.ds(i * T, T)] = idx


def _vq_scores(x, W, off, nb, G=2):
    B, T, C = x.shape
    K = W.shape[0]
    assert nb % G == 0 and off % G == 0
    return pl.pallas_call(
        _vq_tc_body,
        grid=(nb // G,),
        in_specs=[
            pl.BlockSpec((G, T, C), lambda i: (i + off // G, 0, 0)),
            pl.BlockSpec((K, C), lambda i: (0, 0)),
        ],
        out_specs=[
            pl.BlockSpec((nb * T,), lambda i: (0,)),
            pl.BlockSpec((G, 1, 128), lambda i: (i, 0, 0)),
        ],
        out_shape=[
            jax.ShapeDtypeStruct((nb * T,), jnp.int32),
            jax.ShapeDtypeStruct((nb, 1, 128), jnp.float32),
        ],
    )(x, W)


def _sc_gather(W, idx_flat, n_out, row0):
    N = idx_flat.shape[0]
    K, C = W.shape
    info = plsc.get_sparse_core_info()
    nw = info.num_cores * info.num_subcores
    assert N % (8 * nw) == 0 and row0 % 8 == 0, (N, nw, row0)
    b_per_w = N // nw
    mesh = plsc.VectorSubcoreMesh(core_axis_name="c", subcore_axis_name="s")

    @functools.partial(
        pl.kernel, mesh=mesh,
        out_type=jax.ShapeDtypeStruct((n_out, C), jnp.float32),
        scratch_types=[
            pltpu.VMEM((b_per_w,), jnp.int32),
            pltpu.VMEM((b_per_w, C), jnp.float32),
            pltpu.SemaphoreType.DMA,
        ],
    )
    def gather_k(table_hbm, idx_hbm, out_hbm, idx_v, rows_v, sem):
        wid = lax.axis_index("s") * info.num_cores + lax.axis_index("c")
        base = wid * b_per_w
        pltpu.sync_copy(idx_hbm.at[pl.ds(base, b_per_w)], idx_v)
        pltpu.async_copy(table_hbm.at[idx_v], rows_v, sem).wait()
        pltpu.sync_copy(rows_v, out_hbm.at[pl.ds(row0 + base, b_per_w)])

    return gather_k(W, idx_flat)


def kernel(x, W):
    B, T, C = x.shape
    h = B // 2
    idx_a, loss_a = _vq_scores(x, W, 0, h)
    q_full = _sc_gather(W, idx_a, B * T, 0)
    idx_b, loss_b = _vq_scores(x, W, h, B - h)
    q_b = _sc_gather(W, idx_b, (B - h) * T, 0)
    quant = lax.dynamic_update_slice(q_full, q_b, (h * T, 0))
    loss = jnp.concatenate([loss_a, loss_b], axis=0)[:, 0, 0]
    quant_st = quant.reshape(B, T, C)
    indices = jnp.concatenate([idx_a, idx_b], axis=0).reshape(B, T)
    return quant_st, loss, loss, indices
